# baseline (speedup 1.0000x reference)
import functools

import jax
import jax.numpy as jnp
from jax import lax
from jax.experimental import pallas as pl
from jax.experimental.pallas import tpu as pltpu
from jax.experimental.pallas import tpu_sc as plsc

_N_NODES = 10000
_N_EDGES = 320000
_L = 16
_NS = 16
_NC = 1
_HIST_PAD = 20480
_OUT_OFF = 10240
_SLICE = _HIST_PAD // _NS
_E_HIST = _N_EDGES // _NS
_E_GATH = _N_EDGES // _NS
_HC = _E_HIST // 2

_mesh = plsc.VectorSubcoreMesh(core_axis_name="c", subcore_axis_name="s",
                               num_cores=_NC)


@functools.partial(
    pl.kernel,
    mesh=_mesh,
    out_type=jax.ShapeDtypeStruct((_N_EDGES,), jnp.float32),
    scratch_types=[
        pltpu.VMEM((_HIST_PAD,), jnp.float32),
        pltpu.VMEM((_HC,), jnp.int32),
        pltpu.VMEM((_HC,), jnp.int32),
        pltpu.VMEM((_E_GATH,), jnp.int32),
        pltpu.VMEM((_E_GATH,), jnp.int32),
        pltpu.VMEM((_E_GATH,), jnp.float32),
        pltpu.VMEM((_SLICE,), jnp.float32),
        pltpu.VMEM((_SLICE,), jnp.float32),
        pltpu.VMEM((_SLICE,), jnp.float32),
        pltpu.VMEM_SHARED((_NS * _HIST_PAD,), jnp.float32),
        pltpu.SemaphoreType.DMA,
        pltpu.SemaphoreType.DMA,
        pltpu.SemaphoreType.DMA,
        pltpu.SemaphoreType.DMA,
    ],
    compiler_params=pltpu.CompilerParams(needs_layout_passes=False,
                                         use_tc_tiling_on_sc=False),
)
def _sds_kernel(ei_hbm, pei_hbm, out_hbm,
                hist, buf_a, buf_b, tail_buf, head_buf, out_buf,
                slice_a, slice_b, acc_buf, partials,
                sem_a, sem_b, sem_tail, sem_head):
    s = lax.axis_index("s")

    zeros = jnp.zeros((_L,), jnp.float32)
    ones = jnp.ones((_L,), jnp.float32)

    hbase = s * _E_HIST
    gbase = s * _E_GATH
    cp_tail = pltpu.async_copy(ei_hbm.at[1, pl.ds(gbase, _E_GATH)],
                               tail_buf, sem_tail)
    cp_head = pltpu.async_copy(ei_hbm.at[0, pl.ds(gbase, _E_GATH)],
                               head_buf, sem_head)

    chunks = [(1, 0, 0), (1, _HC, 0),
              (0, 0, _OUT_OFF), (0, _HC, _OUT_OFF)]
    bufs = [buf_a, buf_b]
    sems = [sem_a, sem_b]

    def start_chunk(k):
        row, off, _ = chunks[k]
        return pltpu.async_copy(pei_hbm.at[row, pl.ds(hbase + off, _HC)],
                                bufs[k % 2], sems[k % 2])

    cps = [start_chunk(0), start_chunk(1)]

    @plsc.parallel_loop(0, _HIST_PAD, step=_L, unroll=16)
    def zero_hist(i):
        hist[pl.ds(i, _L)] = zeros

    @plsc.parallel_loop(0, _SLICE, step=_L, unroll=16)
    def zero_acc(i):
        acc_buf[pl.ds(i, _L)] = zeros

    with jax.named_scope("ph_scat"):
        for k in range(4):
            cps[k].wait()
            buf = bufs[k % 2]
            voff = chunks[k][2]

            @plsc.parallel_loop(0, _HC, step=_L, unroll=25)
            def scat(i, buf=buf, voff=voff):
                v = buf[pl.ds(i, _L)] + voff
                plsc.addupdate_scatter(hist, [v], ones)

            if k + 2 < 4:
                cps.append(start_chunk(k + 2))

    with jax.named_scope("ph_pub"):
        pltpu.sync_copy(hist, partials.at[pl.ds(s * _HIST_PAD, _HIST_PAD)])
        plsc.subcore_barrier()

    sbufs = [slice_a, slice_b]

    def start_slice(t):
        return pltpu.async_copy(
            partials.at[pl.ds(t * _HIST_PAD + s * _SLICE, _SLICE)],
            sbufs[t % 2], sems[t % 2])

    with jax.named_scope("ph_red"):
        rcps = [start_slice(0), start_slice(1)]
        for t in range(_NS):
            rcps[t].wait()
            sb = sbufs[t % 2]

            @plsc.parallel_loop(0, _SLICE, step=_L, unroll=16)
            def red_vec(i, sb=sb):
                sl = pl.ds(i, _L)
                acc_buf[sl] = acc_buf[sl] + sb[sl]

            if t + 2 < _NS:
                rcps.append(start_slice(t + 2))

    with jax.named_scope("ph_comb"):
        pltpu.sync_copy(acc_buf, partials.at[pl.ds(s * _SLICE, _SLICE)])
        plsc.subcore_barrier()
        pltpu.sync_copy(partials.at[pl.ds(0, _HIST_PAD)], hist)

    with jax.named_scope("ph_gath"):
        cp_tail.wait()
        cp_head.wait()

        @plsc.parallel_loop(0, _E_GATH, step=_L, unroll=25)
        def gath(i):
            sl = pl.ds(i, _L)
            a = plsc.load_gather(hist, [tail_buf[sl]])
            b = plsc.load_gather(hist, [head_buf[sl] + _OUT_OFF])
            out_buf[sl] = (a + b) * jnp.float32(0.5)

        pltpu.sync_copy(out_buf, out_hbm.at[pl.ds(gbase, _E_GATH)])


def kernel(z, edge_index, pos_edge_index):
    del z
    return _sds_kernel(edge_index, pos_edge_index)


# speedup vs baseline: 1.0023x; 1.0023x over previous
import functools

import jax
import jax.numpy as jnp
from jax import lax
from jax.experimental import pallas as pl
from jax.experimental.pallas import tpu as pltpu
from jax.experimental.pallas import tpu_sc as plsc

_N_NODES = 10000
_N_EDGES = 320000
_L = 16
_NS = 16
_NC = 1
_HIST_PAD = 20480
_OUT_OFF = 10240
_SLICE = _HIST_PAD // _NS
_E_HIST = _N_EDGES // _NS
_E_GATH = _N_EDGES // _NS
_HC = _E_HIST // 2

_mesh = plsc.VectorSubcoreMesh(core_axis_name="c", subcore_axis_name="s",
                               num_cores=_NC)


@functools.partial(
    pl.kernel,
    mesh=_mesh,
    out_type=jax.ShapeDtypeStruct((_N_EDGES,), jnp.float32),
    scratch_types=[
        pltpu.VMEM((_HIST_PAD,), jnp.float32),
        pltpu.VMEM((_HC,), jnp.int32),
        pltpu.VMEM((_HC,), jnp.int32),
        pltpu.VMEM((_E_GATH,), jnp.int32),
        pltpu.VMEM((_E_GATH,), jnp.int32),
        pltpu.VMEM((_E_GATH,), jnp.float32),
        pltpu.VMEM((_SLICE,), jnp.float32),
        pltpu.VMEM((_SLICE,), jnp.float32),
        pltpu.VMEM((_SLICE,), jnp.float32),
        pltpu.VMEM_SHARED((_NS * _HIST_PAD,), jnp.float32),
        pltpu.SemaphoreType.DMA,
        pltpu.SemaphoreType.DMA,
        pltpu.SemaphoreType.DMA,
        pltpu.SemaphoreType.DMA,
    ],
    compiler_params=pltpu.CompilerParams(needs_layout_passes=False,
                                         use_tc_tiling_on_sc=False),
)
def _sds_kernel(ei_hbm, pei_hbm, out_hbm,
                hist, buf_a, buf_b, tail_buf, head_buf, out_buf,
                slice_a, slice_b, acc_buf, partials,
                sem_a, sem_b, sem_tail, sem_head):
    s = lax.axis_index("s")

    zeros = jnp.zeros((_L,), jnp.float32)
    ones = jnp.ones((_L,), jnp.float32)

    hbase = s * _E_HIST
    gbase = s * _E_GATH
    cp_tail = pltpu.async_copy(ei_hbm.at[1, pl.ds(gbase, _E_GATH)],
                               tail_buf, sem_tail)
    cp_head = pltpu.async_copy(ei_hbm.at[0, pl.ds(gbase, _E_GATH)],
                               head_buf, sem_head)

    chunks = [(1, 0, 0), (1, _HC, 0),
              (0, 0, _OUT_OFF), (0, _HC, _OUT_OFF)]
    bufs = [buf_a, buf_b]
    sems = [sem_a, sem_b]

    def start_chunk(k):
        row, off, _ = chunks[k]
        return pltpu.async_copy(pei_hbm.at[row, pl.ds(hbase + off, _HC)],
                                bufs[k % 2], sems[k % 2])

    cps = [start_chunk(0), start_chunk(1)]

    @plsc.parallel_loop(0, _HIST_PAD, step=_L, unroll=16)
    def zero_hist(i):
        hist[pl.ds(i, _L)] = zeros

    @plsc.parallel_loop(0, _SLICE, step=_L, unroll=16)
    def zero_acc(i):
        acc_buf[pl.ds(i, _L)] = zeros

    for k in range(4):
        cps[k].wait()
        buf = bufs[k % 2]
        voff = chunks[k][2]

        @plsc.parallel_loop(0, _HC, step=_L, unroll=25)
        def scat(i, buf=buf, voff=voff):
            v = buf[pl.ds(i, _L)] + voff
            plsc.addupdate_scatter(hist, [v], ones)

        if k + 2 < 4:
            cps.append(start_chunk(k + 2))

    pltpu.sync_copy(hist, partials.at[pl.ds(s * _HIST_PAD, _HIST_PAD)])
    plsc.subcore_barrier()

    sbufs = [slice_a, slice_b]

    def start_slice(t):
        return pltpu.async_copy(
            partials.at[pl.ds(t * _HIST_PAD + s * _SLICE, _SLICE)],
            sbufs[t % 2], sems[t % 2])

    rcps = [start_slice(0), start_slice(1)]
    for t in range(_NS):
        rcps[t].wait()
        sb = sbufs[t % 2]

        @plsc.parallel_loop(0, _SLICE, step=_L, unroll=16)
        def red_vec(i, sb=sb):
            sl = pl.ds(i, _L)
            acc_buf[sl] = acc_buf[sl] + sb[sl]

        if t + 2 < _NS:
            rcps.append(start_slice(t + 2))

    pltpu.sync_copy(acc_buf, partials.at[pl.ds(s * _SLICE, _SLICE)])
    plsc.subcore_barrier()
    pltpu.sync_copy(partials.at[pl.ds(0, _HIST_PAD)], hist)

    cp_tail.wait()
    cp_head.wait()

    ocps = []
    for w in range(2):

        @plsc.parallel_loop(w * _HC, (w + 1) * _HC, step=_L, unroll=25)
        def gath(i):
            sl = pl.ds(i, _L)
            a = plsc.load_gather(hist, [tail_buf[sl]])
            b = plsc.load_gather(hist, [head_buf[sl] + _OUT_OFF])
            out_buf[sl] = (a + b) * jnp.float32(0.5)

        ocps.append(pltpu.async_copy(out_buf.at[pl.ds(w * _HC, _HC)],
                                     out_hbm.at[pl.ds(gbase + w * _HC, _HC)],
                                     sems[w]))

    ocps[0].wait()
    ocps[1].wait()


def kernel(z, edge_index, pos_edge_index):
    del z
    return _sds_kernel(edge_index, pos_edge_index)


# speedup vs baseline: 1.0416x; 1.0392x over previous
import functools

import jax
import jax.numpy as jnp
from jax import lax
from jax.experimental import pallas as pl
from jax.experimental.pallas import tpu as pltpu
from jax.experimental.pallas import tpu_sc as plsc

_N_NODES = 10000
_N_EDGES = 320000
_L = 16
_NS = 16
_NC = 1
_HIST_PAD = 20480
_OUT_OFF = 10240
_SLICE = _HIST_PAD // _NS
_E_HIST = _N_EDGES // _NS
_E_GATH = _N_EDGES // _NS
_HC = _E_HIST // 2

_mesh = plsc.VectorSubcoreMesh(core_axis_name="c", subcore_axis_name="s",
                               num_cores=_NC)


@functools.partial(
    pl.kernel,
    mesh=_mesh,
    out_type=jax.ShapeDtypeStruct((_N_EDGES,), jnp.float32),
    scratch_types=[
        pltpu.VMEM((_HIST_PAD,), jnp.float32),
        pltpu.VMEM((_HC,), jnp.int32),
        pltpu.VMEM((_HC,), jnp.int32),
        pltpu.VMEM((_E_GATH,), jnp.int32),
        pltpu.VMEM((_E_GATH,), jnp.int32),
        pltpu.VMEM((_E_GATH,), jnp.float32),
        pltpu.VMEM((_SLICE,), jnp.float32),
        pltpu.VMEM((_SLICE,), jnp.float32),
        pltpu.VMEM((_SLICE,), jnp.float32),
        pltpu.VMEM_SHARED((_NS * _HIST_PAD,), jnp.float32),
        pltpu.SemaphoreType.DMA,
        pltpu.SemaphoreType.DMA,
        pltpu.SemaphoreType.DMA,
        pltpu.SemaphoreType.DMA,
    ],
    compiler_params=pltpu.CompilerParams(needs_layout_passes=False,
                                         use_tc_tiling_on_sc=False),
)
def _sds_kernel(ei_hbm, pei_hbm, out_hbm,
                hist, buf_a, buf_b, tail_buf, head_buf, out_buf,
                slice_a, slice_b, acc_buf, partials,
                sem_a, sem_b, sem_tail, sem_head):
    s = lax.axis_index("s")

    zeros = jnp.zeros((_L,), jnp.float32)
    ones = jnp.ones((_L,), jnp.float32)

    hbase = s * _E_HIST
    gbase = s * _E_GATH

    chunks = [(1, 0, 0), (1, _HC, 0),
              (0, 0, _OUT_OFF), (0, _HC, _OUT_OFF)]
    bufs = [buf_a, buf_b]
    sems = [sem_a, sem_b]

    def start_chunk(k):
        row, off, _ = chunks[k]
        return pltpu.async_copy(pei_hbm.at[row, pl.ds(hbase + off, _HC)],
                                bufs[k % 2], sems[k % 2])

    cps = [start_chunk(0), start_chunk(1)]

    @plsc.parallel_loop(0, _HIST_PAD, step=_L, unroll=16)
    def zero_hist(i):
        hist[pl.ds(i, _L)] = zeros

    @plsc.parallel_loop(0, _SLICE, step=_L, unroll=16)
    def zero_acc(i):
        acc_buf[pl.ds(i, _L)] = zeros

    for k in range(4):
        cps[k].wait()
        buf = bufs[k % 2]
        voff = chunks[k][2]

        @plsc.parallel_loop(0, _HC, step=_L, unroll=25)
        def scat(i, buf=buf, voff=voff):
            v = buf[pl.ds(i, _L)] + voff
            plsc.addupdate_scatter(hist, [v], ones)

        if k + 2 < 4:
            cps.append(start_chunk(k + 2))

    cp_tail = pltpu.async_copy(ei_hbm.at[1, pl.ds(gbase, _E_GATH)],
                               tail_buf, sem_tail)
    cp_head = pltpu.async_copy(ei_hbm.at[0, pl.ds(gbase, _E_GATH)],
                               head_buf, sem_head)

    pltpu.sync_copy(hist, partials.at[pl.ds(s * _HIST_PAD, _HIST_PAD)])
    plsc.subcore_barrier()

    sbufs = [slice_a, slice_b]

    def start_slice(t):
        return pltpu.async_copy(
            partials.at[pl.ds(t * _HIST_PAD + s * _SLICE, _SLICE)],
            sbufs[t % 2], sems[t % 2])

    rcps = [start_slice(0), start_slice(1)]
    for t in range(_NS):
        rcps[t].wait()
        sb = sbufs[t % 2]

        @plsc.parallel_loop(0, _SLICE, step=_L, unroll=16)
        def red_vec(i, sb=sb):
            sl = pl.ds(i, _L)
            acc_buf[sl] = acc_buf[sl] + sb[sl]

        if t + 2 < _NS:
            rcps.append(start_slice(t + 2))

    pltpu.sync_copy(acc_buf, partials.at[pl.ds(s * _SLICE, _SLICE)])
    plsc.subcore_barrier()
    pltpu.sync_copy(partials.at[pl.ds(0, _HIST_PAD)], hist)

    cp_tail.wait()
    cp_head.wait()

    ocps = []
    for w in range(2):

        @plsc.parallel_loop(w * _HC, (w + 1) * _HC, step=_L, unroll=25)
        def gath(i):
            sl = pl.ds(i, _L)
            a = plsc.load_gather(hist, [tail_buf[sl]])
            b = plsc.load_gather(hist, [head_buf[sl] + _OUT_OFF])
            out_buf[sl] = (a + b) * jnp.float32(0.5)

        ocps.append(pltpu.async_copy(out_buf.at[pl.ds(w * _HC, _HC)],
                                     out_hbm.at[pl.ds(gbase + w * _HC, _HC)],
                                     sems[w]))

    ocps[0].wait()
    ocps[1].wait()


def kernel(z, edge_index, pos_edge_index):
    del z
    return _sds_kernel(edge_index, pos_edge_index)
